# baseline (speedup 1.0000x reference)
import functools

import jax
import jax.numpy as jnp
from jax import lax
from jax.experimental import pallas as pl
from jax.experimental.pallas import tpu as pltpu
from jax.experimental.pallas import tpu_sc as plsc

VOCAB = 1000000
DIM = 64
BATCH = 16384

NC = 2
NS = 16
NW = NC * NS
B_PER_W = BATCH // NW


def _sc_gather(user, item, embed_user, embed_item):
  mesh = plsc.VectorSubcoreMesh(
      core_axis_name="c", subcore_axis_name="s",
      num_cores=NC, num_subcores=NS)

  @functools.partial(
      pl.kernel,
      out_type=(
          jax.ShapeDtypeStruct((BATCH, DIM), jnp.float32),
          jax.ShapeDtypeStruct((BATCH, DIM), jnp.float32),
      ),
      mesh=mesh,
      scratch_types=[
          pltpu.VMEM((B_PER_W,), jnp.int32),
          pltpu.VMEM((B_PER_W,), jnp.int32),
          pltpu.SemaphoreType.DMA,
      ],
  )
  def k(u_hbm, i_hbm, eu_hbm, ei_hbm, vu_out, vi_out,
        idx_u, idx_i, sem):
    wid = lax.axis_index("s") * NC + lax.axis_index("c")
    base = wid * B_PER_W
    pltpu.sync_copy(u_hbm.at[pl.ds(base, B_PER_W)], idx_u)
    pltpu.sync_copy(i_hbm.at[pl.ds(base, B_PER_W)], idx_i)

    def body(g, carry):
      base_k = g * 16
      iuv = idx_u[pl.ds(base_k, 16)]
      iiv = idx_i[pl.ds(base_k, 16)]
      for j in range(16):
        kk = base + base_k + j
        pltpu.async_copy(eu_hbm.at[pl.ds(iuv[j], 1)],
                         vu_out.at[pl.ds(kk, 1)], sem)
        pltpu.async_copy(ei_hbm.at[pl.ds(iiv[j], 1)],
                         vi_out.at[pl.ds(kk, 1)], sem)
      return carry

    lax.fori_loop(0, B_PER_W // 16, body, 0)

    pltpu.make_async_copy(eu_hbm.at[pl.ds(0, B_PER_W)],
                          vu_out.at[pl.ds(base, B_PER_W)], sem).wait()
    pltpu.make_async_copy(ei_hbm.at[pl.ds(0, B_PER_W)],
                          vi_out.at[pl.ds(base, B_PER_W)], sem).wait()

  return k(user, item, embed_user, embed_item)


def _tc_body(u_ref, i_ref, w1u_ref, w1i_ref, b1_ref, w2_ref, b2_ref,
             w3_ref, b3_ref, out_ref):
  u = u_ref[...]
  v = i_ref[...]
  pred = jnp.sum(u * v, axis=1)
  h = jnp.maximum(
      u @ w1u_ref[...] + v @ w1i_ref[...] + b1_ref[...], 0.0)
  h = jnp.maximum(h @ w2_ref[...] + b2_ref[...], 0.0)
  noise = jnp.maximum(h @ w3_ref[...] + b3_ref[...], 0.0)
  out_ref[...] = pred + noise[:, 0]


def kernel(user, item, embed_user, embed_item, W1, b1, W2, b2, W3, b3):
  vec_u, vec_i = _sc_gather(user.astype(jnp.int32), item.astype(jnp.int32),
                            embed_user, embed_item)

  w1u = W1[:DIM]
  w1i = W1[DIM:]
  out = pl.pallas_call(
      _tc_body,
      out_shape=jax.ShapeDtypeStruct((BATCH,), jnp.float32),
  )(vec_u, vec_i, w1u, w1i, b1, W2, b2, W3, b3)
  return out


# speedup vs baseline: 1.6787x; 1.6787x over previous
import functools

import jax
import jax.numpy as jnp
from jax import lax
from jax.experimental import pallas as pl
from jax.experimental.pallas import tpu as pltpu
from jax.experimental.pallas import tpu_sc as plsc

VOCAB = 1000000
DIM = 64
BATCH = 16384

NC = 2
NS = 16
NW = NC * NS
B_PER_W = BATCH // NW
WAVE = 128
NWAVE = B_PER_W // WAVE


def _sc_gather(user, item, embed_user, embed_item):
  mesh = plsc.VectorSubcoreMesh(
      core_axis_name="c", subcore_axis_name="s",
      num_cores=NC, num_subcores=NS)

  @functools.partial(
      pl.kernel,
      out_type=(
          jax.ShapeDtypeStruct((BATCH, DIM), jnp.float32),
          jax.ShapeDtypeStruct((BATCH, DIM), jnp.float32),
      ),
      mesh=mesh,
      scratch_types=[
          pltpu.VMEM((B_PER_W,), jnp.int32),
          pltpu.VMEM((B_PER_W,), jnp.int32),
          pltpu.VMEM((2, WAVE, DIM), jnp.float32),
          pltpu.VMEM((2, WAVE, DIM), jnp.float32),
          [pltpu.SemaphoreType.DMA, pltpu.SemaphoreType.DMA],
          [pltpu.SemaphoreType.DMA, pltpu.SemaphoreType.DMA],
          [pltpu.SemaphoreType.DMA, pltpu.SemaphoreType.DMA],
          [pltpu.SemaphoreType.DMA, pltpu.SemaphoreType.DMA],
      ],
  )
  def k(u_hbm, i_hbm, eu_hbm, ei_hbm, vu_out, vi_out,
        idx_u, idx_i, rows_u, rows_i, gsem_u, gsem_i, wsem_u, wsem_i):
    wid = lax.axis_index("s") * NC + lax.axis_index("c")
    base = wid * B_PER_W
    pltpu.sync_copy(u_hbm.at[pl.ds(base, B_PER_W)], idx_u)
    pltpu.sync_copy(i_hbm.at[pl.ds(base, B_PER_W)], idx_i)

    def fire(g, slot):
      def gi(q, c):
        off = g * WAVE + q * 16
        iuv = idx_u[pl.ds(off, 16)]
        iiv = idx_i[pl.ds(off, 16)]
        for j in range(16):
          dst = pl.ds(q * 16 + j, 1)
          pltpu.async_copy(eu_hbm.at[pl.ds(iuv[j], 1)],
                           rows_u.at[slot].at[dst], gsem_u[slot])
          pltpu.async_copy(ei_hbm.at[pl.ds(iiv[j], 1)],
                           rows_i.at[slot].at[dst], gsem_i[slot])
        return c
      lax.fori_loop(0, WAVE // 16, gi, 0)

    def wait_writeback(slot):
      pltpu.make_async_copy(eu_hbm.at[pl.ds(0, WAVE)],
                            vu_out.at[pl.ds(base, WAVE)], wsem_u[slot]).wait()
      pltpu.make_async_copy(ei_hbm.at[pl.ds(0, WAVE)],
                            vi_out.at[pl.ds(base, WAVE)], wsem_i[slot]).wait()

    def drain_and_writeback(g, slot):
      pltpu.make_async_copy(eu_hbm.at[pl.ds(0, WAVE)], rows_u.at[slot],
                            gsem_u[slot]).wait()
      pltpu.make_async_copy(ei_hbm.at[pl.ds(0, WAVE)], rows_i.at[slot],
                            gsem_i[slot]).wait()
      dst = pl.ds(base + g * WAVE, WAVE)
      pltpu.async_copy(rows_u.at[slot], vu_out.at[dst], wsem_u[slot])
      pltpu.async_copy(rows_i.at[slot], vi_out.at[dst], wsem_i[slot])

    fire(0, 0)
    for g in range(NWAVE):
      if g + 1 < NWAVE:
        if g + 1 >= 2:
          wait_writeback((g + 1) % 2)
        fire(g + 1, (g + 1) % 2)
      drain_and_writeback(g, g % 2)
    wait_writeback(0)
    wait_writeback(1)

  return k(user, item, embed_user, embed_item)


def _tc_body(u_ref, i_ref, w1u_ref, w1i_ref, b1_ref, w2_ref, b2_ref,
             w3_ref, b3_ref, out_ref):
  u = u_ref[...]
  v = i_ref[...]
  pred = jnp.sum(u * v, axis=1)
  h = jnp.maximum(
      u @ w1u_ref[...] + v @ w1i_ref[...] + b1_ref[...], 0.0)
  h = jnp.maximum(h @ w2_ref[...] + b2_ref[...], 0.0)
  noise = jnp.maximum(h @ w3_ref[...] + b3_ref[...], 0.0)
  out_ref[...] = pred + noise[:, 0]


def kernel(user, item, embed_user, embed_item, W1, b1, W2, b2, W3, b3):
  vec_u, vec_i = _sc_gather(user.astype(jnp.int32), item.astype(jnp.int32),
                            embed_user, embed_item)

  w1u = W1[:DIM]
  w1i = W1[DIM:]
  out = pl.pallas_call(
      _tc_body,
      out_shape=jax.ShapeDtypeStruct((BATCH,), jnp.float32),
  )(vec_u, vec_i, w1u, w1i, b1, W2, b2, W3, b3)
  return out
